# baseline (speedup 1.0000x reference)
import functools

import jax
import jax.numpy as jnp
from jax import lax
from jax.experimental import pallas as pl
from jax.experimental.pallas import tpu as pltpu
from jax.experimental.pallas import tpu_sc as plsc

_BANK = 16
_ALPHA = 0.1
_SIM_HIGH = 0.8
_SIM_LOW = 0.3
_LANES = 16


_POOL_RB = 8


def _pool_body(m_ref, v_ref, o_ref):
    for r in range(_POOL_RB):
        v = v_ref[r]
        m = m_ref[r]
        hw = v.shape[0]
        msum = jnp.sum(m)
        denom = jnp.maximum(msum, 1e-6)
        lhs = jnp.concatenate([m, jnp.ones_like(m)], axis=0)
        sums = lax.dot_general(lhs, v, (((1,), (0,)), ((), ())))
        s_row = sums[0:1]
        fb_row = sums[1:2] * (1.0 / hw)
        cand = jnp.where(denom <= 1e-5, fb_row, s_row / denom)
        n2 = jnp.sum(cand * cand)
        o_ref[r] = cand / jnp.maximum(jnp.sqrt(n2), 1e-12)


def _pool(v4, m3, row0, rows):
    _, HW, C = v4.shape
    rb = _POOL_RB
    ob = row0 // rb
    return pl.pallas_call(
        _pool_body,
        grid=(rows // rb,),
        in_specs=[
            pl.BlockSpec((rb, 1, HW), lambda i: (i + ob, 0, 0)),
            pl.BlockSpec((rb, HW, C), lambda i: (i + ob, 0, 0)),
        ],
        out_specs=pl.BlockSpec((rb, 1, C), lambda i: (i, 0, 0)),
        out_shape=jax.ShapeDtypeStruct((rows, 1, C), jnp.float32),
    )(m3, v4)


_LAST = 15


def _splat_last(x):
    idx = jnp.full((_LANES, 1), _LAST, jnp.int32)
    dnums = lax.GatherDimensionNumbers(
        offset_dims=(), collapsed_slice_dims=(0,), start_index_map=(0,))
    return lax.gather(x, idx, dnums, (1,),
                      mode=lax.GatherScatterMode.PROMISE_IN_BOUNDS)


def _hsum(x):
    return _splat_last(plsc.cumsum(x))


def _hmax(x):
    return _splat_last(plsc.cummax(x))


def _nrsqrt(x):
    x = jnp.maximum(x, 1e-30)
    i = plsc.bitcast(x, jnp.int32)
    i = jnp.int32(0x5F3759DF) - lax.shift_right_logical(i, 1)
    y = plsc.bitcast(i, jnp.float32)
    for _ in range(3):
        y = y * (1.5 - 0.5 * x * y * y)
    return jnp.minimum(y, 1e12)


def _bandit_row(i, C, cand_v, proto_v, valid_v, pack_v, pg, fg):
    K = _BANK
    nch = C // _LANES
    cj = [cand_v[i, pl.ds(j * _LANES, _LANES)] for j in range(nch)]
    csq = cj[0] * cj[0]
    for j in range(1, nch):
        csq = csq + cj[j] * cj[j]
    candsq = _hsum(csq)

    iota = lax.iota(jnp.int32, _LANES)
    simnum = jnp.zeros((_LANES,), jnp.float32)
    norms2 = jnp.zeros((_LANES,), jnp.float32)
    for k in range(K):
        p0 = proto_v[i, pl.ds(k * C, _LANES)]
        acc_s = p0 * cj[0]
        acc_n = p0 * p0
        for j in range(1, nch):
            pkj = proto_v[i, pl.ds(k * C + j * _LANES, _LANES)]
            acc_s = acc_s + pkj * cj[j]
            acc_n = acc_n + pkj * pkj
        simnum = jnp.where(iota == k, _hsum(acc_s), simnum)
        norms2 = jnp.where(iota == k, _hsum(acc_n), norms2)

    vb = valid_v[i, :] > 0.5
    sim = simnum * _nrsqrt(norms2)
    simm = jnp.where(vb, sim, -1e9)
    best = _hmax(simm)
    best_idx = plsc.all_reduce_ffs(simm == best)
    nvalid = plsc.all_reduce_population_count(vb)
    invb = jnp.logical_not(vb)
    ninv = plsc.all_reduce_population_count(invb)
    spawn = jnp.where(ninv > 0, plsc.all_reduce_ffs(invb), 0)
    any_valid = nvalid > 0
    refine = any_valid & (best >= _SIM_HIGH)
    write = jnp.logical_not(any_valid) | (any_valid & (best <= _SIM_LOW))
    slot = jnp.where(refine, best_idx, spawn)
    slot_oh = iota == slot
    dot_old = _hsum(jnp.where(slot_oh, simnum, 0.0))
    n2_old = _hsum(jnp.where(slot_oh, norms2, 0.0))
    one_m = 1.0 - _ALPHA
    b2 = (one_m * one_m) * n2_old \
        + (2.0 * one_m * _ALPHA) * dot_old \
        + (_ALPHA * _ALPHA) * candsq
    invn = _nrsqrt(b2)
    a_old = jnp.where(refine, one_m * invn,
                      jnp.where(write, jnp.zeros_like(invn),
                                jnp.ones_like(invn)))
    a_cand = jnp.where(refine, _ALPHA * invn,
                       jnp.where(write, jnp.ones_like(invn),
                                 jnp.zeros_like(invn)))
    s2slot = a_old * dot_old + a_cand * candsq
    sim2 = jnp.where(slot_oh, s2slot, simnum)
    vnew = vb | (slot_oh & (refine | write))
    logits = jnp.where(vnew, sim2, -1e9)
    e = jnp.exp(logits - _hmax(logits))
    w = e / _hsum(e)
    wslot = _hsum(jnp.where(slot_oh, w, 0.0))
    w_eff = pg * jnp.where(slot_oh, w * a_old, w)
    beta = pg * wslot * a_cand
    zero = jnp.zeros((_LANES,), jnp.float32)
    pack_v[i, pl.ds(0, _LANES)] = w_eff
    pack_v[i, pl.ds(_LANES, _LANES)] = jnp.where(
        iota == 0, beta, jnp.where(iota == 1, fg, zero))
    for j in range(2, 8):
        pack_v[i, pl.ds(j * _LANES, _LANES)] = zero


def _bandit_sc(cand2, protoflat, validf, pgv, fgv):
    R = cand2.shape[0]
    C = cand2.shape[1]
    mesh = plsc.VectorSubcoreMesh(core_axis_name="c", subcore_axis_name="s")
    rp = R // 32

    @functools.partial(
        pl.kernel,
        out_type=jax.ShapeDtypeStruct((R, 8 * _LANES), jnp.float32),
        mesh=mesh,
        compiler_params=pltpu.CompilerParams(
            needs_layout_passes=False, skip_device_barrier=True),
        scratch_types=[
            pltpu.VMEM((rp, C), jnp.float32),
            pltpu.VMEM((rp, _BANK * C), jnp.float32),
            pltpu.VMEM((rp, _LANES), jnp.float32),
            pltpu.VMEM((_LANES,), jnp.float32),
            pltpu.VMEM((_LANES,), jnp.float32),
            pltpu.VMEM((rp, 8 * _LANES), jnp.float32),
            pltpu.SemaphoreType.DMA,
            pltpu.SemaphoreType.DMA,
            pltpu.SemaphoreType.DMA,
        ],
    )
    def run(cand_hbm, proto_hbm, valid_hbm, pg_hbm, fg_hbm, out_hbm,
            cand_v, proto_v, valid_v, pg_v, fg_v, pack_v, s1, s2, s3):
        wid = lax.axis_index("s") * 2 + lax.axis_index("c")
        r0 = wid * rp
        c1 = pltpu.async_copy(cand_hbm.at[pl.ds(r0, rp)], cand_v, s1)
        c2 = pltpu.async_copy(proto_hbm.at[pl.ds(r0, rp)], proto_v, s2)
        c3 = pltpu.async_copy(valid_hbm.at[pl.ds(r0, rp)], valid_v, s3)
        pltpu.sync_copy(pg_hbm, pg_v)
        pltpu.sync_copy(fg_hbm, fg_v)
        pg = pg_v[...]
        fg = fg_v[...]
        c1.wait()
        c2.wait()
        c3.wait()
        for i in range(rp):
            _bandit_row(i, C, cand_v, proto_v, valid_v, pack_v, pg, fg)
        pltpu.sync_copy(pack_v, out_hbm.at[pl.ds(r0, rp)])

    return run(cand2, protoflat, validf, pgv, fgv)


_MAP_RB = 4


def _map_body(v_ref, f_ref, p_ref, c_ref, pk_ref, o_ref):
    f = f_ref[0]
    for r in range(_MAP_RB):
        v = v_ref[r]
        p = p_ref[r]
        c = c_ref[r]
        pk = pk_ref[r]
        w_eff = pk[:, 0:_BANK]
        beta = pk[:, _BANK:_BANK + 1]
        fg = pk[:, _BANK + 1:_BANK + 2]
        feat = lax.dot_general(w_eff, p, (((1,), (0,)), ((), ())))
        o_ref[r] = fg * (v + f) + (feat + beta * c)


def _map_body_acc(prev_ref, v_ref, f_ref, p_ref, c_ref, pk_ref, o_ref):
    del prev_ref
    _map_body(v_ref, f_ref, p_ref, c_ref, pk_ref, o_ref)


def _mapout(v4, f4, proto3, cand3, pack3, row0, rows, prev=None):
    R, HW, C = v4.shape
    N = R // f4.shape[0]
    rb = _MAP_RB
    ob = row0 // rb
    specs = [
        pl.BlockSpec((rb, HW, C), lambda i: (i + ob, 0, 0)),
        pl.BlockSpec((1, HW, C), lambda i: ((i * rb + row0) // N, 0, 0)),
        pl.BlockSpec((rb, _BANK, C), lambda i: (i, 0, 0)),
        pl.BlockSpec((rb, 1, C), lambda i: (i, 0, 0)),
        pl.BlockSpec((rb, 1, 8 * _LANES), lambda i: (i, 0, 0)),
    ]
    out_spec = pl.BlockSpec((rb, HW, C), lambda i: (i + ob, 0, 0))
    out_shape = jax.ShapeDtypeStruct((R, HW, C), jnp.float32)
    if prev is None:
        return pl.pallas_call(
            _map_body,
            grid=(rows // rb,),
            in_specs=specs,
            out_specs=out_spec,
            out_shape=out_shape,
        )(v4, f4, proto3, cand3, pack3)
    return pl.pallas_call(
        _map_body_acc,
        grid=(rows // rb,),
        in_specs=[pl.BlockSpec((1, 8, 128), lambda i: (0, 0, 0))] + specs,
        out_specs=out_spec,
        out_shape=out_shape,
        input_output_aliases={0: 0},
    )(prev, v4, f4, proto3, cand3, pack3)


def kernel(value_BNCHW, frame_feat_BCHW, mask_BNHW, proto, valid,
           proto_gate, frame_gate):
    B, N, C, H, W = value_BNCHW.shape
    R, HW, K = B * N, H * W, _BANK
    v4 = value_BNCHW.transpose(0, 1, 3, 4, 2).reshape(R, HW, C)
    m3 = mask_BNHW.reshape(R, 1, HW)
    f4 = frame_feat_BCHW.transpose(0, 2, 3, 1).reshape(B, HW, C)

    validf = valid.reshape(R, K).astype(jnp.float32)
    pgv = jnp.full((_LANES,), proto_gate, jnp.float32)
    fgv = jnp.full((_LANES,), frame_gate, jnp.float32)
    pflat = proto.reshape(R, K * C)
    p3 = proto.reshape(R, K, C)
    G = R // 2

    candA = _pool(v4, m3, 0, G)
    candB = _pool(v4, m3, G, G)
    packA = _bandit_sc(candA.reshape(G, C), pflat[:G], validf[:G], pgv, fgv)
    packB = _bandit_sc(candB.reshape(G, C), pflat[G:], validf[G:], pgv, fgv)
    outA = _mapout(v4, f4, p3[:G], candA,
                   packA.reshape(G, 1, 8 * _LANES), 0, G)
    out4 = _mapout(v4, f4, p3[G:], candB,
                   packB.reshape(G, 1, 8 * _LANES), G, G, prev=outA)
    return (out4.reshape(B, N, H, W, C).transpose(0, 1, 4, 2, 3))


# speedup vs baseline: 1.0819x; 1.0819x over previous
import functools

import jax
import jax.numpy as jnp
from jax import lax
from jax.experimental import pallas as pl
from jax.experimental.pallas import tpu as pltpu
from jax.experimental.pallas import tpu_sc as plsc

_BANK = 16
_ALPHA = 0.1
_SIM_HIGH = 0.8
_SIM_LOW = 0.3
_LANES = 16


_POOL_RB = 8


def _pool_body(m_ref, v_ref, o_ref):
    for r in range(_POOL_RB):
        v = v_ref[r]
        m = m_ref[r]
        hw = v.shape[0]
        msum = jnp.sum(m)
        denom = jnp.maximum(msum, 1e-6)
        lhs = jnp.concatenate([m, jnp.ones_like(m)], axis=0)
        sums = lax.dot_general(lhs, v, (((1,), (0,)), ((), ())))
        s_row = sums[0:1]
        fb_row = sums[1:2] * (1.0 / hw)
        cand = jnp.where(denom <= 1e-5, fb_row, s_row / denom)
        n2 = jnp.sum(cand * cand)
        o_ref[r] = cand / jnp.maximum(jnp.sqrt(n2), 1e-12)


def _pool(v4, m3):
    R, HW, C = v4.shape
    rb = _POOL_RB
    return pl.pallas_call(
        _pool_body,
        grid=(R // rb,),
        in_specs=[
            pl.BlockSpec((rb, 1, HW), lambda i: (i, 0, 0)),
            pl.BlockSpec((rb, HW, C), lambda i: (i, 0, 0)),
        ],
        out_specs=pl.BlockSpec((rb, 1, C), lambda i: (i, 0, 0)),
        out_shape=jax.ShapeDtypeStruct((R, 1, C), jnp.float32),
    )(m3, v4)


_LAST = 15


def _splat_last(x):
    idx = jnp.full((_LANES, 1), _LAST, jnp.int32)
    dnums = lax.GatherDimensionNumbers(
        offset_dims=(), collapsed_slice_dims=(0,), start_index_map=(0,))
    return lax.gather(x, idx, dnums, (1,),
                      mode=lax.GatherScatterMode.PROMISE_IN_BOUNDS)


def _hsum(x):
    return _splat_last(plsc.cumsum(x))


def _hmax(x):
    return _splat_last(plsc.cummax(x))


def _nrsqrt(x):
    x = jnp.maximum(x, 1e-30)
    i = plsc.bitcast(x, jnp.int32)
    i = jnp.int32(0x5F3759DF) - lax.shift_right_logical(i, 1)
    y = plsc.bitcast(i, jnp.float32)
    for _ in range(3):
        y = y * (1.5 - 0.5 * x * y * y)
    return jnp.minimum(y, 1e12)


def _bandit_row(i, C, cand_v, proto_v, valid_v, pack_v, pg, fg):
    K = _BANK
    nch = C // _LANES
    cj = [cand_v[i, pl.ds(j * _LANES, _LANES)] for j in range(nch)]
    csq = cj[0] * cj[0]
    for j in range(1, nch):
        csq = csq + cj[j] * cj[j]
    candsq = _hsum(csq)

    iota = lax.iota(jnp.int32, _LANES)
    simnum = jnp.zeros((_LANES,), jnp.float32)
    norms2 = jnp.zeros((_LANES,), jnp.float32)
    for k in range(K):
        p0 = proto_v[i, pl.ds(k * C, _LANES)]
        acc_s = p0 * cj[0]
        acc_n = p0 * p0
        for j in range(1, nch):
            pkj = proto_v[i, pl.ds(k * C + j * _LANES, _LANES)]
            acc_s = acc_s + pkj * cj[j]
            acc_n = acc_n + pkj * pkj
        simnum = jnp.where(iota == k, _hsum(acc_s), simnum)
        norms2 = jnp.where(iota == k, _hsum(acc_n), norms2)

    vb = valid_v[i, :] > 0.5
    sim = simnum * _nrsqrt(norms2)
    simm = jnp.where(vb, sim, -1e9)
    best = _hmax(simm)
    best_idx = plsc.all_reduce_ffs(simm == best)
    nvalid = plsc.all_reduce_population_count(vb)
    invb = jnp.logical_not(vb)
    ninv = plsc.all_reduce_population_count(invb)
    spawn = jnp.where(ninv > 0, plsc.all_reduce_ffs(invb), 0)
    any_valid = nvalid > 0
    refine = any_valid & (best >= _SIM_HIGH)
    write = jnp.logical_not(any_valid) | (any_valid & (best <= _SIM_LOW))
    slot = jnp.where(refine, best_idx, spawn)
    slot_oh = iota == slot
    dot_old = _hsum(jnp.where(slot_oh, simnum, 0.0))
    n2_old = _hsum(jnp.where(slot_oh, norms2, 0.0))
    one_m = 1.0 - _ALPHA
    b2 = (one_m * one_m) * n2_old \
        + (2.0 * one_m * _ALPHA) * dot_old \
        + (_ALPHA * _ALPHA) * candsq
    invn = _nrsqrt(b2)
    a_old = jnp.where(refine, one_m * invn,
                      jnp.where(write, jnp.zeros_like(invn),
                                jnp.ones_like(invn)))
    a_cand = jnp.where(refine, _ALPHA * invn,
                       jnp.where(write, jnp.ones_like(invn),
                                 jnp.zeros_like(invn)))
    s2slot = a_old * dot_old + a_cand * candsq
    sim2 = jnp.where(slot_oh, s2slot, simnum)
    vnew = vb | (slot_oh & (refine | write))
    logits = jnp.where(vnew, sim2, -1e9)
    e = jnp.exp(logits - _hmax(logits))
    w = e / _hsum(e)
    wslot = _hsum(jnp.where(slot_oh, w, 0.0))
    w_eff = pg * jnp.where(slot_oh, w * a_old, w)
    beta = pg * wslot * a_cand
    zero = jnp.zeros((_LANES,), jnp.float32)
    pack_v[i, pl.ds(0, _LANES)] = w_eff
    pack_v[i, pl.ds(_LANES, _LANES)] = jnp.where(
        iota == 0, beta, jnp.where(iota == 1, fg, zero))
    for j in range(2, 8):
        pack_v[i, pl.ds(j * _LANES, _LANES)] = zero


def _bandit_sc(cand2, protoflat, validf, pgv, fgv):
    R = cand2.shape[0]
    C = cand2.shape[1]
    mesh = plsc.VectorSubcoreMesh(core_axis_name="c", subcore_axis_name="s")
    rp = R // 32

    @functools.partial(
        pl.kernel,
        out_type=jax.ShapeDtypeStruct((R, 8 * _LANES), jnp.float32),
        mesh=mesh,
        compiler_params=pltpu.CompilerParams(
            needs_layout_passes=False, skip_device_barrier=True),
        scratch_types=[
            pltpu.VMEM((rp, C), jnp.float32),
            pltpu.VMEM((rp, _BANK * C), jnp.float32),
            pltpu.VMEM((rp, _LANES), jnp.float32),
            pltpu.VMEM((_LANES,), jnp.float32),
            pltpu.VMEM((_LANES,), jnp.float32),
            pltpu.VMEM((rp, 8 * _LANES), jnp.float32),
            pltpu.SemaphoreType.DMA,
            pltpu.SemaphoreType.DMA,
            pltpu.SemaphoreType.DMA,
        ],
    )
    def run(cand_hbm, proto_hbm, valid_hbm, pg_hbm, fg_hbm, out_hbm,
            cand_v, proto_v, valid_v, pg_v, fg_v, pack_v, s1, s2, s3):
        wid = lax.axis_index("s") * 2 + lax.axis_index("c")
        r0 = wid * rp
        c1 = pltpu.async_copy(cand_hbm.at[pl.ds(r0, rp)], cand_v, s1)
        c2 = pltpu.async_copy(proto_hbm.at[pl.ds(r0, rp)], proto_v, s2)
        c3 = pltpu.async_copy(valid_hbm.at[pl.ds(r0, rp)], valid_v, s3)
        pltpu.sync_copy(pg_hbm, pg_v)
        pltpu.sync_copy(fg_hbm, fg_v)
        pg = pg_v[...]
        fg = fg_v[...]
        c1.wait()
        c2.wait()
        c3.wait()
        for i in range(rp):
            _bandit_row(i, C, cand_v, proto_v, valid_v, pack_v, pg, fg)
        pltpu.sync_copy(pack_v, out_hbm.at[pl.ds(r0, rp)])

    return run(cand2, protoflat, validf, pgv, fgv)


_MAP_RB = 4


def _map_body(v_ref, f_ref, p_ref, c_ref, pk_ref, o_ref):
    f = f_ref[0]
    for r in range(_MAP_RB):
        v = v_ref[r]
        p = p_ref[r]
        c = c_ref[r]
        pk = pk_ref[r]
        w_eff = pk[:, 0:_BANK]
        beta = pk[:, _BANK:_BANK + 1]
        fg = pk[:, _BANK + 1:_BANK + 2]
        feat = lax.dot_general(w_eff, p, (((1,), (0,)), ((), ())))
        o_ref[r] = fg * (v + f) + (feat + beta * c)


def _mapout(v4, f4, proto3, cand3, pack3):
    R, HW, C = v4.shape
    N = R // f4.shape[0]
    rb = _MAP_RB
    return pl.pallas_call(
        _map_body,
        grid=(R // rb,),
        in_specs=[
            pl.BlockSpec((rb, HW, C), lambda i: (i, 0, 0)),
            pl.BlockSpec((1, HW, C), lambda i: (i * rb // N, 0, 0)),
            pl.BlockSpec((rb, _BANK, C), lambda i: (i, 0, 0)),
            pl.BlockSpec((rb, 1, C), lambda i: (i, 0, 0)),
            pl.BlockSpec((rb, 1, 8 * _LANES), lambda i: (i, 0, 0)),
        ],
        out_specs=pl.BlockSpec((rb, HW, C), lambda i: (i, 0, 0)),
        out_shape=jax.ShapeDtypeStruct((R, HW, C), jnp.float32),
    )(v4, f4, proto3, cand3, pack3)


def kernel(value_BNCHW, frame_feat_BCHW, mask_BNHW, proto, valid,
           proto_gate, frame_gate):
    B, N, C, H, W = value_BNCHW.shape
    R, HW, K = B * N, H * W, _BANK
    v4 = value_BNCHW.transpose(0, 1, 3, 4, 2).reshape(R, HW, C)
    m3 = mask_BNHW.reshape(R, 1, HW)
    f4 = frame_feat_BCHW.transpose(0, 2, 3, 1).reshape(B, HW, C)

    cand3 = _pool(v4, m3)

    validf = valid.reshape(R, K).astype(jnp.float32)
    pgv = jnp.full((_LANES,), proto_gate, jnp.float32)
    fgv = jnp.full((_LANES,), frame_gate, jnp.float32)
    pack = _bandit_sc(cand3.reshape(R, C), proto.reshape(R, K * C),
                      validf, pgv, fgv)

    out4 = _mapout(v4, f4, proto.reshape(R, K, C), cand3,
                   pack.reshape(R, 1, 8 * _LANES))
    return (out4.reshape(B, N, H, W, C).transpose(0, 1, 4, 2, 3))
